# Initial kernel scaffold; baseline (speedup 1.0000x reference)
import jax
import jax.numpy as jnp
from jax.experimental import pallas as pl


def kernel(x_target, x_context, edge_index_tt, edge_index_ct, W_lin_t, b_lin_t, W_lin_c, b_lin_c, W_self, b_self, W_s2d, b_s2d, W_d2s, b_d2s, W_ct_l, b_ct_l, W_ct_r, W_out, b_out):
    raise NotImplementedError("write your pallas kernel here")


# speedup vs baseline: 4.2536x; 4.2536x over previous
import functools

import jax
import jax.numpy as jnp
from jax import lax
from jax.experimental import pallas as pl
from jax.experimental.pallas import tpu as pltpu
from jax.experimental.pallas import tpu_sc as plsc

F32 = jnp.float32
FCH = 32
EBLK = 512
NSUB = 16
ZROWS = 800


def _tc1_body(xt_in, xc_in, wlt, blt, wlc, blc, wself, bself, ws2d, bs2d,
              wd2s, bd2s, wctl, bctl, wctr, xt_out, pre_out, *chunk_outs):
    xt = jnp.maximum(jnp.dot(xt_in[:], wlt[:], preferred_element_type=F32) + blt[:], 0.0)
    xc = jnp.maximum(jnp.dot(xc_in[:], wlc[:], preferred_element_type=F32) + blc[:], 0.0)
    xt_out[:] = xt
    bias = bself[:] + 0.5 * bs2d[:] + 0.5 * bd2s[:] + bctl[:]
    pre_out[:] = jnp.dot(xt, wself[:] + wctr[:], preferred_element_type=F32) + bias
    ys = (jnp.dot(xt, ws2d[:], preferred_element_type=F32),
          jnp.dot(xt, wd2s[:], preferred_element_type=F32),
          jnp.dot(xc, wctl[:], preferred_element_type=F32))
    for o in range(3):
        for f in range(4):
            chunk_outs[o * 4 + f][:] = ys[o][:, f * FCH:(f + 1) * FCH]


def _tc2_body(pre, xt, t10, t11, t12, t13, t20, t21, t22, t23,
              t30, t31, t32, t33, c1, c2, c3, wout, bout, out):
    T1 = jnp.concatenate([t10[:], t11[:], t12[:], t13[:]], axis=1)
    T2 = jnp.concatenate([t20[:], t21[:], t22[:], t23[:]], axis=1)
    T3 = jnp.concatenate([t30[:], t31[:], t32[:], t33[:]], axis=1)
    d1 = jnp.maximum(c1[:][:, 0:1], 1.0)
    d2 = jnp.maximum(c2[:][:, 0:1], 1.0)
    d3 = jnp.maximum(c3[:][:, 0:1], 1.0)
    h = 0.5 * pre[:] + 0.25 * T1 / d1 + 0.25 * T2 / d2 + 0.5 * T3 / d3
    h = jnp.maximum(h + xt[:], 0.0)
    out[:] = jnp.dot(h, wout[:], preferred_element_type=F32) + bout[:]


def _sc_scatter_call(NT, EPAD, tables, gidxs, sidxs, zeros_h, ones_h):
    Q = EPAD // NSUB
    NB = Q // EBLK
    RPT = (-(-(NT // NSUB) // ZROWS)) * ZROWS
    NTP = RPT * NSUB
    QR = Q // 128
    mesh = plsc.VectorSubcoreMesh(core_axis_name="c", subcore_axis_name="s")

    def body(*refs):
        tbl = refs[0:12]
        gx = refs[12:15]
        sx = refs[15:18]
        zeros_hbm = refs[18]
        ones_hbm = refs[19]
        outs = refs[20:35]
        acc, idxg, idxs, rows, sem = refs[35:40]

        c = lax.axis_index("c")
        s = lax.axis_index("s")

        def do_pass(p, table, g_h, s_h, out_h, is_count):
            @pl.when(c == (p % 2))
            def _():
                for k in range(RPT // ZROWS):
                    pltpu.sync_copy(zeros_hbm, acc.at[pl.ds(s * RPT + k * ZROWS, ZROWS)])
                if is_count:
                    pltpu.sync_copy(ones_hbm, rows)
                plsc.subcore_barrier()

                def block(b, carry):
                    base = s * Q + b * EBLK
                    rowbase = s * QR + b * (EBLK // 128)
                    pltpu.sync_copy(s_h.at[pl.ds(rowbase, EBLK // 128)], idxs)
                    if not is_count:
                        pltpu.sync_copy(g_h.at[pl.ds(base, EBLK)], idxg)
                        pltpu.async_copy(table.at[idxg], rows, sem).wait()

                    def scat(j, carry2):
                        off = pl.multiple_of(j * 128, 128)
                        pltpu.sync_copy(rows.at[pl.ds(off, 128)],
                                        acc.at[idxs.at[j]], add=True)
                        return carry2
                    lax.fori_loop(0, EBLK // 128, scat, 0)
                    return carry
                lax.fori_loop(0, NB, block, 0)
                plsc.subcore_barrier()
                pltpu.sync_copy(acc.at[pl.ds(s * RPT, RPT)],
                                out_h.at[pl.ds(s * RPT, RPT)])

        p = 0
        for o in range(3):
            for f in range(4):
                do_pass(p, tbl[o * 4 + f], gx[o], sx[o], outs[o * 4 + f], False)
                p += 1
        for o in range(3):
            do_pass(p, None, None, sx[o], outs[12 + o], True)
            p += 1

    out_type = [jax.ShapeDtypeStruct((NTP, FCH), F32) for _ in range(15)]
    scratch = [
        pltpu.VMEM_SHARED((NTP, FCH), F32),
        pltpu.VMEM((EBLK,), jnp.int32),
        pltpu.VMEM((EBLK // 128, 128), jnp.int32),
        pltpu.VMEM((EBLK, FCH), F32),
        pltpu.SemaphoreType.DMA,
    ]
    fn = pl.kernel(body, out_type=out_type, mesh=mesh, scratch_types=scratch,
                   compiler_params=pltpu.CompilerParams(use_tc_tiling_on_sc=False))
    return fn(*tables, *gidxs, *sidxs, zeros_h, ones_h)


def kernel(x_target, x_context, edge_index_tt, edge_index_ct,
           W_lin_t, b_lin_t, W_lin_c, b_lin_c,
           W_self, b_self, W_s2d, b_s2d, W_d2s, b_d2s,
           W_ct_l, b_ct_l, W_ct_r, W_out, b_out):
    NT, D = x_target.shape
    H = W_lin_t.shape[1]
    E = edge_index_tt.shape[1]
    BM = 1000
    grid = (NT // BM,)

    b2 = lambda b: b.reshape(1, H)
    row_spec = pl.BlockSpec((BM, D), lambda i: (i, 0))
    w_spec = pl.BlockSpec((D, H), lambda i: (0, 0))
    b_spec = pl.BlockSpec((1, H), lambda i: (0, 0))
    chunk_spec = pl.BlockSpec((BM, FCH), lambda i: (i, 0))

    tc1 = pl.pallas_call(
        _tc1_body,
        grid=grid,
        in_specs=[row_spec, row_spec, w_spec, b_spec, w_spec, b_spec,
                  w_spec, b_spec, w_spec, b_spec, w_spec, b_spec,
                  w_spec, b_spec, w_spec],
        out_specs=[row_spec, row_spec] + [chunk_spec] * 12,
        out_shape=[jax.ShapeDtypeStruct((NT, H), F32)] * 2
        + [jax.ShapeDtypeStruct((NT, FCH), F32)] * 12,
    )
    xt, pre, *chunks = tc1(x_target, x_context, W_lin_t, b2(b_lin_t),
                           W_lin_c, b2(b_lin_c), W_self, b2(b_self),
                           W_s2d, b2(b_s2d), W_d2s, b2(b_d2s),
                           W_ct_l, b2(b_ct_l), W_ct_r)

    EPAD = ((E + NSUB * EBLK - 1) // (NSUB * EBLK)) * (NSUB * EBLK)
    npad = EPAD - E
    pad_g = jnp.zeros((npad,), jnp.int32)
    pad_s = jnp.full((npad,), NT, jnp.int32)

    def gpad(a):
        return jnp.concatenate([a, pad_g])

    def spad(a):
        return jnp.concatenate([a, pad_s]).reshape(EPAD // 128, 128)

    gidxs = [gpad(edge_index_tt[0]), gpad(edge_index_tt[1]), gpad(edge_index_ct[0])]
    sidxs = [spad(edge_index_tt[1]), spad(edge_index_tt[0]), spad(edge_index_ct[1])]
    zeros_h = jnp.zeros((ZROWS, FCH), F32)
    ones_h = jnp.ones((EBLK, FCH), F32)

    sc_outs = _sc_scatter_call(NT, EPAD, chunks, gidxs, sidxs, zeros_h, ones_h)
    tsums = sc_outs[0:12]
    counts = sc_outs[12:15]

    tc2 = pl.pallas_call(
        _tc2_body,
        grid=grid,
        in_specs=[row_spec, row_spec] + [chunk_spec] * 12 + [chunk_spec] * 3
        + [w_spec, b_spec],
        out_specs=row_spec,
        out_shape=jax.ShapeDtypeStruct((NT, H), F32),
    )
    return tc2(pre, xt, *tsums, *counts, W_out, b2(b_out))


# speedup vs baseline: 4.9930x; 1.1738x over previous
import jax
import jax.numpy as jnp
from jax import lax
from jax.experimental import pallas as pl
from jax.experimental.pallas import tpu as pltpu
from jax.experimental.pallas import tpu_sc as plsc

F32 = jnp.float32
FCH = 32
EBLK = 768
NSUB = 16
ZROWS = 800


def _tc1_body(xt_in, xc_in, wlt, blt, wlc, blc, wself, bself, bs2d, bd2s,
              bctl, wctr, xt_out, xc_out, pre_out):
    xt = jnp.maximum(jnp.dot(xt_in[:], wlt[:], preferred_element_type=F32) + blt[:], 0.0)
    xc = jnp.maximum(jnp.dot(xc_in[:], wlc[:], preferred_element_type=F32) + blc[:], 0.0)
    xt_out[:] = xt
    xc_out[:] = xc
    bias = bself[:] + 0.5 * bs2d[:] + 0.5 * bd2s[:] + bctl[:]
    pre_out[:] = jnp.dot(xt, wself[:] + wctr[:], preferred_element_type=F32) + bias


def _tc2_body(pre, xt, m1, m2, m3, cnt, ws2d, wd2s, wctl, wout, bout, out):
    d1 = jnp.maximum(cnt[:][:, 0:1], 1.0)
    d2 = jnp.maximum(cnt[:][:, 32:33], 1.0)
    d3 = jnp.maximum(cnt[:][:, 64:65], 1.0)
    h = (0.5 * pre[:]
         + 0.25 * jnp.dot(m1[:] / d1, ws2d[:], preferred_element_type=F32)
         + 0.25 * jnp.dot(m2[:] / d2, wd2s[:], preferred_element_type=F32)
         + 0.5 * jnp.dot(m3[:] / d3, wctl[:], preferred_element_type=F32))
    h = jnp.maximum(h + xt[:], 0.0)
    out[:] = jnp.dot(h, wout[:], preferred_element_type=F32) + bout[:]


def _sc_scatter_call(NT, EPAD, xt32, xc32, gidxs, sidxs, zeros_h, ones_h):
    Q = EPAD // NSUB
    NB = Q // EBLK
    RPT = (-(-(NT // NSUB) // ZROWS)) * ZROWS
    NTP = RPT * NSUB
    QR = Q // 128
    mesh = plsc.VectorSubcoreMesh(core_axis_name="c", subcore_axis_name="s")

    def body(*refs):
        xt_t = refs[0]
        xc_t = refs[1]
        gx = refs[2:14]
        sx = refs[14:17]
        zeros_hbm = refs[17]
        ones_hbm = refs[18]
        outs = refs[19:34]
        acc, idxg, idxs, rows, sem = refs[34:39]

        c = lax.axis_index("c")
        s = lax.axis_index("s")

        def do_pass(p, table, g_h, s_h, out_h, is_count):
            @pl.when(c == (p % 2))
            def _():
                for k in range(RPT // ZROWS):
                    pltpu.sync_copy(zeros_hbm, acc.at[pl.ds(s * RPT + k * ZROWS, ZROWS)])
                if is_count:
                    pltpu.sync_copy(ones_hbm, rows)
                plsc.subcore_barrier()

                def block(b, carry):
                    base = s * Q + b * EBLK
                    rowbase = s * QR + b * (EBLK // 128)
                    pltpu.sync_copy(s_h.at[pl.ds(rowbase, EBLK // 128)], idxs)
                    if not is_count:
                        pltpu.sync_copy(g_h.at[pl.ds(base, EBLK)], idxg)
                        pltpu.async_copy(table.at[idxg], rows, sem).wait()

                    def scat(j, carry2):
                        off = pl.multiple_of(j * 128, 128)
                        pltpu.sync_copy(rows.at[pl.ds(off, 128)],
                                        acc.at[idxs.at[j]], add=True)
                        return carry2
                    lax.fori_loop(0, EBLK // 128, scat, 0)
                    return carry
                lax.fori_loop(0, NB, block, 0)
                plsc.subcore_barrier()
                pltpu.sync_copy(acc.at[pl.ds(s * RPT, RPT)],
                                out_h.at[pl.ds(s * RPT, RPT)])

        tables = (xt_t, xt_t, xc_t)
        p = 0
        for o in range(3):
            for f in range(4):
                do_pass(p, tables[o], gx[o * 4 + f], sx[o], outs[o * 4 + f], False)
                p += 1
        for o in range(3):
            do_pass(p, None, None, sx[o], outs[12 + o], True)
            p += 1

    out_type = [jax.ShapeDtypeStruct((NTP, FCH), F32) for _ in range(15)]
    scratch = [
        pltpu.VMEM_SHARED((NTP, FCH), F32),
        pltpu.VMEM((EBLK,), jnp.int32),
        pltpu.VMEM((EBLK // 128, 128), jnp.int32),
        pltpu.VMEM((EBLK, FCH), F32),
        pltpu.SemaphoreType.DMA,
    ]
    fn = pl.kernel(body, out_type=out_type, mesh=mesh, scratch_types=scratch,
                   compiler_params=pltpu.CompilerParams(use_tc_tiling_on_sc=False))
    return fn(xt32, xc32, *gidxs, *sidxs, zeros_h, ones_h)


def kernel(x_target, x_context, edge_index_tt, edge_index_ct,
           W_lin_t, b_lin_t, W_lin_c, b_lin_c,
           W_self, b_self, W_s2d, b_s2d, W_d2s, b_d2s,
           W_ct_l, b_ct_l, W_ct_r, W_out, b_out):
    NT, D = x_target.shape
    H = W_lin_t.shape[1]
    E = edge_index_tt.shape[1]
    BM = 1000
    grid = (NT // BM,)

    b2 = lambda b: b.reshape(1, H)
    row_spec = pl.BlockSpec((BM, D), lambda i: (i, 0))
    w_spec = pl.BlockSpec((D, H), lambda i: (0, 0))
    b_spec = pl.BlockSpec((1, H), lambda i: (0, 0))

    tc1 = pl.pallas_call(
        _tc1_body,
        grid=grid,
        in_specs=[row_spec, row_spec, w_spec, b_spec, w_spec, b_spec,
                  w_spec, b_spec, b_spec, b_spec, b_spec, w_spec],
        out_specs=[row_spec] * 3,
        out_shape=[jax.ShapeDtypeStruct((NT, H), F32)] * 3,
    )
    xt, xc, pre = tc1(x_target, x_context, W_lin_t, b2(b_lin_t),
                      W_lin_c, b2(b_lin_c), W_self, b2(b_self),
                      b2(b_s2d), b2(b_d2s), b2(b_ct_l), W_ct_r)

    EPAD = ((E + NSUB * EBLK - 1) // (NSUB * EBLK)) * (NSUB * EBLK)
    npad = EPAD - E
    pad_g = jnp.zeros((npad,), jnp.int32)
    pad_s = jnp.full((npad,), NT, jnp.int32)

    gbase = [jnp.concatenate([edge_index_tt[0], pad_g]) * 4,
             jnp.concatenate([edge_index_tt[1], pad_g]) * 4,
             jnp.concatenate([edge_index_ct[0], pad_g]) * 4]
    gidxs = [gbase[o] + f for o in range(3) for f in range(4)]

    def spad(a):
        return jnp.concatenate([a, pad_s]).reshape(EPAD // 128, 128)

    sidxs = [spad(edge_index_tt[1]), spad(edge_index_tt[0]), spad(edge_index_ct[1])]
    zeros_h = jnp.zeros((ZROWS, FCH), F32)
    ones_h = jnp.ones((EBLK, FCH), F32)

    xt32 = xt.reshape(4 * NT, FCH)
    xc32 = xc.reshape(4 * NT, FCH)
    sc_outs = _sc_scatter_call(NT, EPAD, xt32, xc32, gidxs, sidxs, zeros_h, ones_h)

    ms = [jnp.concatenate([sc_outs[o * 4 + f][:NT] for f in range(4)], axis=1)
          for o in range(3)]
    cnt = jnp.concatenate([sc_outs[12][:NT], sc_outs[13][:NT],
                           sc_outs[14][:NT], sc_outs[12][:NT]], axis=1)

    tc2 = pl.pallas_call(
        _tc2_body,
        grid=grid,
        in_specs=[row_spec] * 6 + [w_spec, w_spec, w_spec, w_spec, b_spec],
        out_specs=row_spec,
        out_shape=jax.ShapeDtypeStruct((NT, H), F32),
    )
    return tc2(pre, xt, *ms, cnt, W_s2d, W_d2s, W_ct_l, W_out, b2(b_out))


# speedup vs baseline: 5.2609x; 1.0536x over previous
import jax
import jax.numpy as jnp
from jax import lax
from jax.experimental import pallas as pl
from jax.experimental.pallas import tpu as pltpu
from jax.experimental.pallas import tpu_sc as plsc

F32 = jnp.float32
FCH = 32
EBLK = 384
NSUB = 16
ZROWS = 800


def _tc1_body(xt_in, xc_in, wlt, blt, wlc, blc, wself, bself, bs2d, bd2s,
              bctl, wctr, xt_out, xc_out, pre_out):
    xt = jnp.maximum(jnp.dot(xt_in[:], wlt[:], preferred_element_type=F32) + blt[:], 0.0)
    xc = jnp.maximum(jnp.dot(xc_in[:], wlc[:], preferred_element_type=F32) + blc[:], 0.0)
    xt_out[:] = xt
    xc_out[:] = xc
    bias = bself[:] + 0.5 * bs2d[:] + 0.5 * bd2s[:] + bctl[:]
    pre_out[:] = jnp.dot(xt, wself[:] + wctr[:], preferred_element_type=F32) + bias


def _tc2_body(pre, xt, m1, m2, m3, cnt, ws2d, wd2s, wctl, wout, bout, out):
    d1 = jnp.maximum(cnt[:][:, 0:1], 1.0)
    d2 = jnp.maximum(cnt[:][:, 32:33], 1.0)
    d3 = jnp.maximum(cnt[:][:, 64:65], 1.0)
    h = (0.5 * pre[:]
         + 0.25 * jnp.dot(m1[:] / d1, ws2d[:], preferred_element_type=F32)
         + 0.25 * jnp.dot(m2[:] / d2, wd2s[:], preferred_element_type=F32)
         + 0.5 * jnp.dot(m3[:] / d3, wctl[:], preferred_element_type=F32))
    h = jnp.maximum(h + xt[:], 0.0)
    out[:] = jnp.dot(h, wout[:], preferred_element_type=F32) + bout[:]


def _sc_scatter_call(NT, EPAD, xt32, xc32, gidxs, sidxs, zeros_h, ones_h):
    Q = EPAD // NSUB
    NB = Q // EBLK
    RPT = (-(-(NT // NSUB) // ZROWS)) * ZROWS
    NTP = RPT * NSUB
    QR = Q // 128
    mesh = plsc.VectorSubcoreMesh(core_axis_name="c", subcore_axis_name="s")

    def body(*refs):
        xt_t = refs[0]
        xc_t = refs[1]
        gx = refs[2:14]
        sx = refs[14:17]
        zeros_hbm = refs[17]
        ones_hbm = refs[18]
        outs = refs[19:34]
        acc = refs[34]
        idxg = refs[35:37]
        idxs = refs[37:39]
        rows = refs[39:41]
        sems = refs[41:43]

        c = lax.axis_index("c")
        s = lax.axis_index("s")
        NSB = EBLK // 128

        def do_pass(p, table, g_h, s_h, out_h, is_count):
            @pl.when(c == (p % 2))
            def _():
                for k in range(RPT // ZROWS):
                    pltpu.sync_copy(zeros_hbm, acc.at[pl.ds(s * RPT + k * ZROWS, ZROWS)])

                def load_idx(b, buf):
                    rowbase = s * QR + b * NSB
                    pltpu.sync_copy(s_h.at[pl.ds(rowbase, NSB)], idxs[buf])
                    if not is_count:
                        pltpu.sync_copy(g_h.at[pl.ds(s * Q + b * EBLK, EBLK)],
                                        idxg[buf])

                def start_gather(buf):
                    pltpu.async_copy(table.at[idxg[buf]], rows[buf], sems[buf])

                def finish_gather(buf):
                    pltpu.make_async_copy(table.at[idxg[buf]], rows[buf],
                                          sems[buf]).wait()

                def scatter(buf):
                    def scat(j, carry2):
                        off = pl.multiple_of(j * 128, 128)
                        pltpu.sync_copy(rows[buf].at[pl.ds(off, 128)],
                                        acc.at[idxs[buf].at[j]], add=True)
                        return carry2
                    lax.fori_loop(0, NSB, scat, 0)

                if is_count:
                    pltpu.sync_copy(ones_hbm, rows[0])
                    plsc.subcore_barrier()

                    def cblock(b, carry):
                        load_idx(b, 0)
                        scatter(0)
                        return carry
                    lax.fori_loop(0, NB, cblock, 0)
                else:
                    plsc.subcore_barrier()
                    load_idx(0, 0)
                    start_gather(0)

                    def block2(i, carry):
                        b = i * 2
                        load_idx(b + 1, 1)
                        start_gather(1)
                        finish_gather(0)
                        scatter(0)

                        @pl.when(i + 1 < NB // 2)
                        def _():
                            load_idx(b + 2, 0)
                            start_gather(0)
                        finish_gather(1)
                        scatter(1)
                        return carry
                    lax.fori_loop(0, NB // 2, block2, 0)
                plsc.subcore_barrier()
                pltpu.sync_copy(acc.at[pl.ds(s * RPT, RPT)],
                                out_h.at[pl.ds(s * RPT, RPT)])

        tables = (xt_t, xt_t, xc_t)
        p = 0
        for o in range(3):
            for f in range(4):
                do_pass(p, tables[o], gx[o * 4 + f], sx[o], outs[o * 4 + f], False)
                p += 1
        for o in range(3):
            do_pass(p, None, None, sx[o], outs[12 + o], True)
            p += 1

    out_type = [jax.ShapeDtypeStruct((NTP, FCH), F32) for _ in range(15)]
    scratch = [
        pltpu.VMEM_SHARED((NTP, FCH), F32),
        pltpu.VMEM((EBLK,), jnp.int32),
        pltpu.VMEM((EBLK,), jnp.int32),
        pltpu.VMEM((EBLK // 128, 128), jnp.int32),
        pltpu.VMEM((EBLK // 128, 128), jnp.int32),
        pltpu.VMEM((EBLK, FCH), F32),
        pltpu.VMEM((EBLK, FCH), F32),
        pltpu.SemaphoreType.DMA,
        pltpu.SemaphoreType.DMA,
    ]
    fn = pl.kernel(body, out_type=out_type, mesh=mesh, scratch_types=scratch,
                   compiler_params=pltpu.CompilerParams(use_tc_tiling_on_sc=False))
    return fn(xt32, xc32, *gidxs, *sidxs, zeros_h, ones_h)


def kernel(x_target, x_context, edge_index_tt, edge_index_ct,
           W_lin_t, b_lin_t, W_lin_c, b_lin_c,
           W_self, b_self, W_s2d, b_s2d, W_d2s, b_d2s,
           W_ct_l, b_ct_l, W_ct_r, W_out, b_out):
    NT, D = x_target.shape
    H = W_lin_t.shape[1]
    E = edge_index_tt.shape[1]
    BM = 1000
    grid = (NT // BM,)

    b2 = lambda b: b.reshape(1, H)
    row_spec = pl.BlockSpec((BM, D), lambda i: (i, 0))
    w_spec = pl.BlockSpec((D, H), lambda i: (0, 0))
    b_spec = pl.BlockSpec((1, H), lambda i: (0, 0))

    tc1 = pl.pallas_call(
        _tc1_body,
        grid=grid,
        in_specs=[row_spec, row_spec, w_spec, b_spec, w_spec, b_spec,
                  w_spec, b_spec, b_spec, b_spec, b_spec, w_spec],
        out_specs=[row_spec] * 3,
        out_shape=[jax.ShapeDtypeStruct((NT, H), F32)] * 3,
    )
    xt, xc, pre = tc1(x_target, x_context, W_lin_t, b2(b_lin_t),
                      W_lin_c, b2(b_lin_c), W_self, b2(b_self),
                      b2(b_s2d), b2(b_d2s), b2(b_ct_l), W_ct_r)

    EPAD = ((E + NSUB * EBLK - 1) // (NSUB * EBLK)) * (NSUB * EBLK)
    npad = EPAD - E
    pad_g = jnp.zeros((npad,), jnp.int32)
    pad_s = jnp.full((npad,), NT, jnp.int32)

    gbase = [jnp.concatenate([edge_index_tt[0], pad_g]) * 4,
             jnp.concatenate([edge_index_tt[1], pad_g]) * 4,
             jnp.concatenate([edge_index_ct[0], pad_g]) * 4]
    gidxs = [gbase[o] + f for o in range(3) for f in range(4)]

    def spad(a):
        return jnp.concatenate([a, pad_s]).reshape(EPAD // 128, 128)

    sidxs = [spad(edge_index_tt[1]), spad(edge_index_tt[0]), spad(edge_index_ct[1])]
    zeros_h = jnp.zeros((ZROWS, FCH), F32)
    ones_h = jnp.ones((EBLK, FCH), F32)

    xt32 = xt.reshape(4 * NT, FCH)
    xc32 = xc.reshape(4 * NT, FCH)
    sc_outs = _sc_scatter_call(NT, EPAD, xt32, xc32, gidxs, sidxs, zeros_h, ones_h)

    ms = [jnp.concatenate([sc_outs[o * 4 + f][:NT] for f in range(4)], axis=1)
          for o in range(3)]
    cnt = jnp.concatenate([sc_outs[12][:NT], sc_outs[13][:NT],
                           sc_outs[14][:NT], sc_outs[12][:NT]], axis=1)

    tc2 = pl.pallas_call(
        _tc2_body,
        grid=grid,
        in_specs=[row_spec] * 6 + [w_spec, w_spec, w_spec, w_spec, b_spec],
        out_specs=row_spec,
        out_shape=jax.ShapeDtypeStruct((NT, H), F32),
    )
    return tc2(pre, xt, *ms, cnt, W_s2d, W_d2s, W_ct_l, W_out, b2(b_out))


# speedup vs baseline: 5.3762x; 1.0219x over previous
import jax
import jax.numpy as jnp
from jax import lax
from jax.experimental import pallas as pl
from jax.experimental.pallas import tpu as pltpu
from jax.experimental.pallas import tpu_sc as plsc

F32 = jnp.float32
FCH = 32
EBLK = 384
NSUB = 16
ZROWS = 800


def _tc1_body(xt_in, xc_in, wlt, blt, wlc, blc, wself, bself, bs2d, bd2s,
              bctl, wctr, xt_out, xc_out, pre_out):
    xt = jnp.maximum(jnp.dot(xt_in[:], wlt[:], preferred_element_type=F32) + blt[:], 0.0)
    xc = jnp.maximum(jnp.dot(xc_in[:], wlc[:], preferred_element_type=F32) + blc[:], 0.0)
    xt_out[:] = xt
    xc_out[:] = xc
    bias = bself[:] + 0.5 * bs2d[:] + 0.5 * bd2s[:] + bctl[:]
    pre_out[:] = jnp.dot(xt, wself[:] + wctr[:], preferred_element_type=F32) + bias


def _tc2_body(pre, xt, m1, m2, m3, cnt, ws2d, wd2s, wctl, wout, bout, out):
    d1 = jnp.maximum(cnt[:][:, 0:1], 1.0)
    d2 = jnp.maximum(cnt[:][:, 32:33], 1.0)
    d3 = jnp.maximum(cnt[:][:, 64:65], 1.0)
    h = (0.5 * pre[:]
         + 0.25 * jnp.dot(m1[:] / d1, ws2d[:], preferred_element_type=F32)
         + 0.25 * jnp.dot(m2[:] / d2, wd2s[:], preferred_element_type=F32)
         + 0.5 * jnp.dot(m3[:] / d3, wctl[:], preferred_element_type=F32))
    h = jnp.maximum(h + xt[:], 0.0)
    out[:] = jnp.dot(h, wout[:], preferred_element_type=F32) + bout[:]


def _sc_scatter_call(NT, EPAD, xt32, xc32, gidxs, sidxs, zeros_h, ones_h):
    Q = EPAD // NSUB
    NB = Q // EBLK
    RPT = (-(-(NT // NSUB) // ZROWS)) * ZROWS
    NTP = RPT * NSUB
    QR = Q // 128
    mesh = plsc.VectorSubcoreMesh(core_axis_name="c", subcore_axis_name="s")

    def body(*refs):
        xt_t = refs[0]
        xc_t = refs[1]
        gx = refs[2:14]
        sx = refs[14:17]
        zeros_hbm = refs[17]
        ones_hbm = refs[18]
        outs = refs[19:34]
        acc = refs[34]
        idxg = refs[35:37]
        idxs = refs[37:39]
        rows = refs[39:41]
        sems = refs[41:43]
        ssems = refs[43:45]

        c = lax.axis_index("c")
        s = lax.axis_index("s")
        NSB = EBLK // 128

        def do_pass(p, table, g_h, s_h, out_h, is_count):
            @pl.when(c == (p % 2))
            def _():
                for k in range(RPT // ZROWS):
                    pltpu.sync_copy(zeros_hbm, acc.at[pl.ds(s * RPT + k * ZROWS, ZROWS)])

                def load_idx(b, buf):
                    rowbase = s * QR + b * NSB
                    pltpu.sync_copy(s_h.at[pl.ds(rowbase, NSB)], idxs[buf])
                    if not is_count:
                        pltpu.sync_copy(g_h.at[pl.ds(s * Q + b * EBLK, EBLK)],
                                        idxg[buf])

                def start_gather(buf):
                    pltpu.async_copy(table.at[idxg[buf]], rows[buf], sems[buf])

                def finish_gather(buf):
                    pltpu.make_async_copy(table.at[idxg[buf]], rows[buf],
                                          sems[buf]).wait()

                def fire_scatters(buf):
                    for j in range(NSB):
                        pltpu.async_copy(rows[buf].at[pl.ds(j * 128, 128)],
                                         acc.at[idxs[buf].at[j]], ssems[buf],
                                         add=True)

                def drain_scatters(buf):
                    for j in range(NSB):
                        pltpu.make_async_copy(rows[buf].at[pl.ds(j * 128, 128)],
                                              acc.at[idxs[buf].at[j]],
                                              ssems[buf]).wait()

                if is_count:
                    pltpu.sync_copy(ones_hbm, rows[0])
                    plsc.subcore_barrier()

                    def cblock(b, carry):
                        load_idx(b, 0)
                        fire_scatters(0)
                        drain_scatters(0)
                        return carry
                    lax.fori_loop(0, NB, cblock, 0)
                else:
                    plsc.subcore_barrier()
                    load_idx(0, 0)
                    start_gather(0)

                    def block2(i, carry):
                        b = i * 2

                        @pl.when(i > 0)
                        def _():
                            drain_scatters(1)
                        load_idx(b + 1, 1)
                        start_gather(1)
                        finish_gather(0)
                        fire_scatters(0)
                        finish_gather(1)
                        fire_scatters(1)

                        @pl.when(i + 1 < NB // 2)
                        def _():
                            drain_scatters(0)
                            load_idx(b + 2, 0)
                            start_gather(0)
                        return carry
                    lax.fori_loop(0, NB // 2, block2, 0)
                    drain_scatters(0)
                    drain_scatters(1)
                plsc.subcore_barrier()
                pltpu.sync_copy(acc.at[pl.ds(s * RPT, RPT)],
                                out_h.at[pl.ds(s * RPT, RPT)])

        tables = (xt_t, xt_t, xc_t)
        p = 0
        for o in range(3):
            for f in range(4):
                do_pass(p, tables[o], gx[o * 4 + f], sx[o], outs[o * 4 + f], False)
                p += 1
        for o in range(3):
            do_pass(p, None, None, sx[o], outs[12 + o], True)
            p += 1

    out_type = [jax.ShapeDtypeStruct((NTP, FCH), F32) for _ in range(15)]
    scratch = [
        pltpu.VMEM_SHARED((NTP, FCH), F32),
        pltpu.VMEM((EBLK,), jnp.int32),
        pltpu.VMEM((EBLK,), jnp.int32),
        pltpu.VMEM((EBLK // 128, 128), jnp.int32),
        pltpu.VMEM((EBLK // 128, 128), jnp.int32),
        pltpu.VMEM((EBLK, FCH), F32),
        pltpu.VMEM((EBLK, FCH), F32),
        pltpu.SemaphoreType.DMA,
        pltpu.SemaphoreType.DMA,
        pltpu.SemaphoreType.DMA,
        pltpu.SemaphoreType.DMA,
    ]
    fn = pl.kernel(body, out_type=out_type, mesh=mesh, scratch_types=scratch,
                   compiler_params=pltpu.CompilerParams(use_tc_tiling_on_sc=False))
    return fn(xt32, xc32, *gidxs, *sidxs, zeros_h, ones_h)


def kernel(x_target, x_context, edge_index_tt, edge_index_ct,
           W_lin_t, b_lin_t, W_lin_c, b_lin_c,
           W_self, b_self, W_s2d, b_s2d, W_d2s, b_d2s,
           W_ct_l, b_ct_l, W_ct_r, W_out, b_out):
    NT, D = x_target.shape
    H = W_lin_t.shape[1]
    E = edge_index_tt.shape[1]
    BM = 1000
    grid = (NT // BM,)

    b2 = lambda b: b.reshape(1, H)
    row_spec = pl.BlockSpec((BM, D), lambda i: (i, 0))
    w_spec = pl.BlockSpec((D, H), lambda i: (0, 0))
    b_spec = pl.BlockSpec((1, H), lambda i: (0, 0))

    tc1 = pl.pallas_call(
        _tc1_body,
        grid=grid,
        in_specs=[row_spec, row_spec, w_spec, b_spec, w_spec, b_spec,
                  w_spec, b_spec, b_spec, b_spec, b_spec, w_spec],
        out_specs=[row_spec] * 3,
        out_shape=[jax.ShapeDtypeStruct((NT, H), F32)] * 3,
    )
    xt, xc, pre = tc1(x_target, x_context, W_lin_t, b2(b_lin_t),
                      W_lin_c, b2(b_lin_c), W_self, b2(b_self),
                      b2(b_s2d), b2(b_d2s), b2(b_ct_l), W_ct_r)

    EPAD = ((E + NSUB * EBLK - 1) // (NSUB * EBLK)) * (NSUB * EBLK)
    npad = EPAD - E
    pad_g = jnp.zeros((npad,), jnp.int32)
    pad_s = jnp.full((npad,), NT, jnp.int32)

    gbase = [jnp.concatenate([edge_index_tt[0], pad_g]) * 4,
             jnp.concatenate([edge_index_tt[1], pad_g]) * 4,
             jnp.concatenate([edge_index_ct[0], pad_g]) * 4]
    gidxs = [gbase[o] + f for o in range(3) for f in range(4)]

    def spad(a):
        return jnp.concatenate([a, pad_s]).reshape(EPAD // 128, 128)

    sidxs = [spad(edge_index_tt[1]), spad(edge_index_tt[0]), spad(edge_index_ct[1])]
    zeros_h = jnp.zeros((ZROWS, FCH), F32)
    ones_h = jnp.ones((EBLK, FCH), F32)

    xt32 = xt.reshape(4 * NT, FCH)
    xc32 = xc.reshape(4 * NT, FCH)
    sc_outs = _sc_scatter_call(NT, EPAD, xt32, xc32, gidxs, sidxs, zeros_h, ones_h)

    ms = [jnp.concatenate([sc_outs[o * 4 + f][:NT] for f in range(4)], axis=1)
          for o in range(3)]
    cnt = jnp.concatenate([sc_outs[12][:NT], sc_outs[13][:NT],
                           sc_outs[14][:NT], sc_outs[12][:NT]], axis=1)

    tc2 = pl.pallas_call(
        _tc2_body,
        grid=grid,
        in_specs=[row_spec] * 6 + [w_spec, w_spec, w_spec, w_spec, b_spec],
        out_specs=row_spec,
        out_shape=jax.ShapeDtypeStruct((NT, H), F32),
    )
    return tc2(pre, xt, *ms, cnt, W_s2d, W_d2s, W_ct_l, W_out, b2(b_out))


# speedup vs baseline: 6.2679x; 1.1659x over previous
import jax
import jax.numpy as jnp
from jax import lax
from jax.experimental import pallas as pl
from jax.experimental.pallas import tpu as pltpu
from jax.experimental.pallas import tpu_sc as plsc

F32 = jnp.float32
FCH = 32
EBLK = 384
NSUB = 16
ZROWS = 800
NSB = EBLK // 128


def _tc1_body(xt_in, xc_in, wlt, blt, wlc, blc, wself, bself, bs2d, bd2s,
              bctl, wctr, xt_out, xc_out, pre_out):
    xt = jnp.maximum(jnp.dot(xt_in[:], wlt[:], preferred_element_type=F32) + blt[:], 0.0)
    xc = jnp.maximum(jnp.dot(xc_in[:], wlc[:], preferred_element_type=F32) + blc[:], 0.0)
    xt_out[:] = xt
    xc_out[:] = xc
    bias = bself[:] + 0.5 * bs2d[:] + 0.5 * bd2s[:] + bctl[:]
    pre_out[:] = jnp.dot(xt, wself[:] + wctr[:], preferred_element_type=F32) + bias


def _tc2_body(pre, xt, m1, m2, m3, cnt, ws2d, wd2s, wctl, wout, bout, out):
    d1 = jnp.maximum(cnt[:][:, 0:1], 1.0)
    d2 = jnp.maximum(cnt[:][:, 32:33], 1.0)
    d3 = jnp.maximum(cnt[:][:, 64:65], 1.0)
    h = (0.5 * pre[:]
         + 0.25 * jnp.dot(m1[:] / d1, ws2d[:], preferred_element_type=F32)
         + 0.25 * jnp.dot(m2[:] / d2, wd2s[:], preferred_element_type=F32)
         + 0.5 * jnp.dot(m3[:] / d3, wctl[:], preferred_element_type=F32))
    h = jnp.maximum(h + xt[:], 0.0)
    out[:] = jnp.dot(h, wout[:], preferred_element_type=F32) + bout[:]


def _zero_acc(acc, zeros_hbm, zsem, s, RPT):
    for k in range(RPT // ZROWS):
        pltpu.async_copy(zeros_hbm, acc.at[pl.ds(s * RPT + k * ZROWS, ZROWS)], zsem)
    for k in range(RPT // ZROWS):
        pltpu.make_async_copy(zeros_hbm, acc.at[pl.ds(s * RPT + k * ZROWS, ZROWS)],
                              zsem).wait()


def _sc_count_call(NT, NTP, RPT, EPAD, sidxs, zeros_h, ones_h):
    Q = EPAD // NSUB
    NB = Q // EBLK
    QR = Q // 128
    mesh = plsc.VectorSubcoreMesh(core_axis_name="c", subcore_axis_name="s")

    def body(s1, s2, s3, zeros_hbm, ones_hbm, o1, o2, o3, acc, idxs, rows,
             ssem, zsem):
        c = lax.axis_index("c")
        s = lax.axis_index("s")
        pltpu.sync_copy(ones_hbm, rows)

        for p, (s_h, out_h) in enumerate(((s1, o1), (s2, o2), (s3, o3))):
            @pl.when(c == (p % 2))
            def _():
                _zero_acc(acc, zeros_hbm, zsem, s, RPT)
                plsc.subcore_barrier()

                def cblock(b, carry):
                    pltpu.sync_copy(s_h.at[pl.ds(s * QR + b * NSB, NSB)], idxs)
                    for j in range(NSB):
                        pltpu.async_copy(rows.at[pl.ds(j * 128, 128)],
                                         acc.at[idxs.at[j]], ssem, add=True)
                    for j in range(NSB):
                        pltpu.make_async_copy(rows.at[pl.ds(j * 128, 128)],
                                              acc.at[idxs.at[j]], ssem).wait()
                    return carry
                lax.fori_loop(0, NB, cblock, 0)
                plsc.subcore_barrier()
                pltpu.sync_copy(acc.at[pl.ds(s * RPT, RPT)],
                                out_h.at[pl.ds(s * RPT, RPT)])

    out_type = [jax.ShapeDtypeStruct((NTP, FCH), F32) for _ in range(3)]
    scratch = [
        pltpu.VMEM_SHARED((NTP, FCH), F32),
        pltpu.VMEM((NSB, 128), jnp.int32),
        pltpu.VMEM((EBLK, FCH), F32),
        pltpu.SemaphoreType.DMA,
        pltpu.SemaphoreType.DMA,
    ]
    fn = pl.kernel(body, out_type=out_type, mesh=mesh, scratch_types=scratch,
                   compiler_params=pltpu.CompilerParams(use_tc_tiling_on_sc=False))
    return fn(*sidxs, zeros_h, ones_h)


def _sc_scatter_call(NT, NTP, RPT, EPAD, xt32, xc32, gidxs, sidxs, zeros_h):
    Q = EPAD // NSUB
    NB = Q // EBLK
    QR = Q // 128
    mesh = plsc.VectorSubcoreMesh(core_axis_name="c", subcore_axis_name="s")

    def body(*refs):
        xt_t = refs[0]
        xc_t = refs[1]
        gx = refs[2:14]
        sx = refs[14:17]
        zeros_hbm = refs[17]
        outs = refs[18:30]
        acc = refs[30]
        idxg = refs[31:33]
        idxs = refs[33:35]
        rows = refs[35:37]
        sems = refs[37:39]
        ssems = refs[39:41]
        zsem = refs[41]

        c = lax.axis_index("c")
        s = lax.axis_index("s")

        def do_pass(p, table, g_h, s_h, out_h):
            @pl.when(c == (p % 2))
            def _():
                _zero_acc(acc, zeros_hbm, zsem, s, RPT)

                def load_idx(b, buf):
                    pltpu.sync_copy(s_h.at[pl.ds(s * QR + b * NSB, NSB)],
                                    idxs[buf])
                    pltpu.sync_copy(g_h.at[pl.ds(s * Q + b * EBLK, EBLK)],
                                    idxg[buf])

                def start_gather(buf):
                    pltpu.async_copy(table.at[idxg[buf]], rows[buf], sems[buf])

                def finish_gather(buf):
                    pltpu.make_async_copy(table.at[idxg[buf]], rows[buf],
                                          sems[buf]).wait()

                def fire_scatters(buf):
                    for j in range(NSB):
                        pltpu.async_copy(rows[buf].at[pl.ds(j * 128, 128)],
                                         acc.at[idxs[buf].at[j]], ssems[buf],
                                         add=True)

                def drain_scatters(buf):
                    for j in range(NSB):
                        pltpu.make_async_copy(rows[buf].at[pl.ds(j * 128, 128)],
                                              acc.at[idxs[buf].at[j]],
                                              ssems[buf]).wait()

                plsc.subcore_barrier()
                load_idx(0, 0)
                start_gather(0)

                def block2(i, carry):
                    b = i * 2

                    @pl.when(i > 0)
                    def _():
                        drain_scatters(1)
                    load_idx(b + 1, 1)
                    start_gather(1)
                    finish_gather(0)
                    fire_scatters(0)
                    finish_gather(1)
                    fire_scatters(1)

                    @pl.when(i + 1 < NB // 2)
                    def _():
                        drain_scatters(0)
                        load_idx(b + 2, 0)
                        start_gather(0)
                    return carry
                lax.fori_loop(0, NB // 2, block2, 0)
                drain_scatters(0)
                drain_scatters(1)
                plsc.subcore_barrier()
                pltpu.sync_copy(acc.at[pl.ds(s * RPT, RPT)],
                                out_h.at[pl.ds(s * RPT, RPT)])

        tables = (xt_t, xt_t, xc_t)
        p = 0
        for o in range(3):
            for f in range(4):
                do_pass(p, tables[o], gx[o * 4 + f], sx[o], outs[o * 4 + f])
                p += 1

    out_type = [jax.ShapeDtypeStruct((NTP, FCH), F32) for _ in range(12)]
    scratch = [
        pltpu.VMEM_SHARED((NTP, FCH), F32),
        pltpu.VMEM((EBLK,), jnp.int32),
        pltpu.VMEM((EBLK,), jnp.int32),
        pltpu.VMEM((NSB, 128), jnp.int32),
        pltpu.VMEM((NSB, 128), jnp.int32),
        pltpu.VMEM((EBLK, FCH), F32),
        pltpu.VMEM((EBLK, FCH), F32),
        pltpu.SemaphoreType.DMA,
        pltpu.SemaphoreType.DMA,
        pltpu.SemaphoreType.DMA,
        pltpu.SemaphoreType.DMA,
        pltpu.SemaphoreType.DMA,
    ]
    fn = pl.kernel(body, out_type=out_type, mesh=mesh, scratch_types=scratch,
                   compiler_params=pltpu.CompilerParams(use_tc_tiling_on_sc=False))
    return fn(xt32, xc32, *gidxs, *sidxs, zeros_h)


def kernel(x_target, x_context, edge_index_tt, edge_index_ct,
           W_lin_t, b_lin_t, W_lin_c, b_lin_c,
           W_self, b_self, W_s2d, b_s2d, W_d2s, b_d2s,
           W_ct_l, b_ct_l, W_ct_r, W_out, b_out):
    NT, D = x_target.shape
    H = W_lin_t.shape[1]
    E = edge_index_tt.shape[1]
    BM = 1000
    grid = (NT // BM,)
    RPT = (-(-(NT // NSUB) // ZROWS)) * ZROWS
    NTP = RPT * NSUB

    EPAD = ((E + NSUB * EBLK - 1) // (NSUB * EBLK)) * (NSUB * EBLK)
    npad = EPAD - E
    spread = jnp.arange(npad, dtype=jnp.int32) % 1024
    pad_g = spread * 4
    pad_s = NT + spread

    gbase = [jnp.concatenate([edge_index_tt[0] * 4, pad_g]),
             jnp.concatenate([edge_index_tt[1] * 4, pad_g]),
             jnp.concatenate([edge_index_ct[0] * 4, pad_g])]
    gidxs = [gbase[o] + f for o in range(3) for f in range(4)]

    def spad(a):
        return jnp.concatenate([a, pad_s]).reshape(EPAD // 128, 128)

    sidxs = [spad(edge_index_tt[1]), spad(edge_index_tt[0]), spad(edge_index_ct[1])]
    zeros_h = jnp.zeros((ZROWS, FCH), F32)
    ones_h = jnp.ones((EBLK, FCH), F32)

    counts = _sc_count_call(NT, NTP, RPT, EPAD, sidxs, zeros_h, ones_h)

    b2 = lambda b: b.reshape(1, H)
    row_spec = pl.BlockSpec((BM, D), lambda i: (i, 0))
    w_spec = pl.BlockSpec((D, H), lambda i: (0, 0))
    b_spec = pl.BlockSpec((1, H), lambda i: (0, 0))

    tc1 = pl.pallas_call(
        _tc1_body,
        grid=grid,
        in_specs=[row_spec, row_spec, w_spec, b_spec, w_spec, b_spec,
                  w_spec, b_spec, b_spec, b_spec, b_spec, w_spec],
        out_specs=[row_spec] * 3,
        out_shape=[jax.ShapeDtypeStruct((NT, H), F32)] * 3,
    )
    xt, xc, pre = tc1(x_target, x_context, W_lin_t, b2(b_lin_t),
                      W_lin_c, b2(b_lin_c), W_self, b2(b_self),
                      b2(b_s2d), b2(b_d2s), b2(b_ct_l), W_ct_r)

    xt32 = xt.reshape(4 * NT, FCH)
    xc32 = xc.reshape(4 * NT, FCH)
    sc_outs = _sc_scatter_call(NT, NTP, RPT, EPAD, xt32, xc32, gidxs, sidxs,
                               zeros_h)

    ms = [jnp.concatenate([sc_outs[o * 4 + f][:NT] for f in range(4)], axis=1)
          for o in range(3)]
    cnt = jnp.concatenate([counts[0][:NT], counts[1][:NT],
                           counts[2][:NT], counts[0][:NT]], axis=1)

    tc2 = pl.pallas_call(
        _tc2_body,
        grid=grid,
        in_specs=[row_spec] * 6 + [w_spec, w_spec, w_spec, w_spec, b_spec],
        out_specs=row_spec,
        out_shape=jax.ShapeDtypeStruct((NT, H), F32),
    )
    return tc2(pre, xt, *ms, cnt, W_s2d, W_d2s, W_ct_l, W_out, b2(b_out))


# speedup vs baseline: 7.9553x; 1.2692x over previous
import jax
import jax.numpy as jnp
from jax import lax
from jax.experimental import pallas as pl
from jax.experimental.pallas import tpu as pltpu
from jax.experimental.pallas import tpu_sc as plsc

F32 = jnp.float32
FCH = 32
EBLK = 384
NSUB = 16
ZROWS = 800
NSB = EBLK // 128


def _tc1_body(xt_in, xc_in, wlt, blt, wlc, blc, wself, bself, bs2d, bd2s,
              bctl, wctr, xt_out, xc_out, pre_out):
    xt = jnp.maximum(jnp.dot(xt_in[:], wlt[:], preferred_element_type=F32) + blt[:], 0.0)
    xc = jnp.maximum(jnp.dot(xc_in[:], wlc[:], preferred_element_type=F32) + blc[:], 0.0)
    xt_out[:] = xt
    xc_out[:] = xc
    bias = bself[:] + 0.5 * bs2d[:] + 0.5 * bd2s[:] + bctl[:]
    pre_out[:] = jnp.dot(xt, wself[:] + wctr[:], preferred_element_type=F32) + bias


def _tc2_body(pre, xt, m1, m2, m3, ca, cb, ws2d, wd2s, wctl, wout, bout, out):
    d1 = jnp.maximum(ca[:][:, 0:1] + cb[:][:, 0:1], 1.0)
    d2 = jnp.maximum(ca[:][:, 32:33] + cb[:][:, 32:33], 1.0)
    d3 = jnp.maximum(ca[:][:, 64:65] + cb[:][:, 64:65], 1.0)
    h = (0.5 * pre[:]
         + 0.25 * jnp.dot(m1[:] / d1, ws2d[:], preferred_element_type=F32)
         + 0.25 * jnp.dot(m2[:] / d2, wd2s[:], preferred_element_type=F32)
         + 0.5 * jnp.dot(m3[:] / d3, wctl[:], preferred_element_type=F32))
    h = jnp.maximum(h + xt[:], 0.0)
    out[:] = jnp.dot(h, wout[:], preferred_element_type=F32) + bout[:]


def _zero_acc(acc, zeros_hbm, zsem, s, RPT):
    for k in range(RPT // ZROWS):
        pltpu.async_copy(zeros_hbm, acc.at[pl.ds(s * RPT + k * ZROWS, ZROWS)], zsem)
    for k in range(RPT // ZROWS):
        pltpu.make_async_copy(zeros_hbm, acc.at[pl.ds(s * RPT + k * ZROWS, ZROWS)],
                              zsem).wait()


def _sc_count_call(NT, NTP, RPT, EPAD, sidxs, zeros_h, ones_h):
    EH = EPAD // 2
    Q = EH // NSUB
    NB = Q // EBLK
    QR = Q // 128
    mesh = plsc.VectorSubcoreMesh(core_axis_name="c", subcore_axis_name="s")

    def body(s1, s2, s3, zeros_hbm, ones_hbm, oa, ob, acc, idxs, rows,
             ssem, zsem):
        c = lax.axis_index("c")
        s = lax.axis_index("s")
        pltpu.sync_copy(ones_hbm, rows)

        for p, s_h in enumerate((s1, s2, s3)):
            _zero_acc(acc, zeros_hbm, zsem, s, RPT)
            plsc.subcore_barrier()

            def cblock(b, carry):
                rowbase = c * (EH // 128) + s * QR + b * NSB
                pltpu.sync_copy(s_h.at[pl.ds(rowbase, NSB)], idxs)
                for j in range(NSB):
                    pltpu.async_copy(rows.at[pl.ds(j * 128, 128)],
                                     acc.at[idxs.at[j]], ssem, add=True)
                for j in range(NSB):
                    pltpu.make_async_copy(rows.at[pl.ds(j * 128, 128)],
                                          acc.at[idxs.at[j]], ssem).wait()
                return carry
            lax.fori_loop(0, NB, cblock, 0)
            plsc.subcore_barrier()

            @pl.when(c == 0)
            def _():
                pltpu.sync_copy(acc.at[pl.ds(s * RPT, RPT)],
                                oa.at[pl.ds(s * RPT, RPT), pl.ds(p * 32, 32)])

            @pl.when(c == 1)
            def _():
                pltpu.sync_copy(acc.at[pl.ds(s * RPT, RPT)],
                                ob.at[pl.ds(s * RPT, RPT), pl.ds(p * 32, 32)])

    out_type = [jax.ShapeDtypeStruct((NTP, 128), F32) for _ in range(2)]
    scratch = [
        pltpu.VMEM_SHARED((NTP, FCH), F32),
        pltpu.VMEM((NSB, 128), jnp.int32),
        pltpu.VMEM((EBLK, FCH), F32),
        pltpu.SemaphoreType.DMA,
        pltpu.SemaphoreType.DMA,
    ]
    fn = pl.kernel(body, out_type=out_type, mesh=mesh, scratch_types=scratch,
                   compiler_params=pltpu.CompilerParams(use_tc_tiling_on_sc=False))
    return fn(*sidxs, zeros_h, ones_h)


def _sc_scatter_call(NT, NTP, RPT, EPAD, xt32, xc32, gidxs, sidxs, zeros_h):
    Q = EPAD // NSUB
    NB = Q // EBLK
    QR = Q // 128
    mesh = plsc.VectorSubcoreMesh(core_axis_name="c", subcore_axis_name="s")

    def body(*refs):
        xt_t = refs[0]
        xc_t = refs[1]
        gx = refs[2:14]
        sx = refs[14:17]
        zeros_hbm = refs[17]
        outs = refs[18:21]
        acc = refs[21]
        idxg = refs[22:24]
        idxs = refs[24:26]
        rows = refs[26:28]
        sems = refs[28:30]
        ssems = refs[30:32]
        zsem = refs[32]

        c = lax.axis_index("c")
        s = lax.axis_index("s")

        def do_pass(p, f, table, g_h, s_h, out_h):
            @pl.when(c == (p % 2))
            def _():
                _zero_acc(acc, zeros_hbm, zsem, s, RPT)

                def load_idx(b, buf):
                    pltpu.sync_copy(s_h.at[pl.ds(s * QR + b * NSB, NSB)],
                                    idxs[buf])
                    pltpu.sync_copy(g_h.at[pl.ds(s * Q + b * EBLK, EBLK)],
                                    idxg[buf])

                def start_gather(buf):
                    pltpu.async_copy(table.at[idxg[buf]], rows[buf], sems[buf])

                def finish_gather(buf):
                    pltpu.make_async_copy(table.at[idxg[buf]], rows[buf],
                                          sems[buf]).wait()

                def fire_scatters(buf):
                    for j in range(NSB):
                        pltpu.async_copy(rows[buf].at[pl.ds(j * 128, 128)],
                                         acc.at[idxs[buf].at[j]], ssems[buf],
                                         add=True)

                def drain_scatters(buf):
                    for j in range(NSB):
                        pltpu.make_async_copy(rows[buf].at[pl.ds(j * 128, 128)],
                                              acc.at[idxs[buf].at[j]],
                                              ssems[buf]).wait()

                plsc.subcore_barrier()
                load_idx(0, 0)
                start_gather(0)

                def block2(i, carry):
                    b = i * 2

                    @pl.when(i > 0)
                    def _():
                        drain_scatters(1)
                    load_idx(b + 1, 1)
                    start_gather(1)
                    finish_gather(0)
                    fire_scatters(0)
                    finish_gather(1)
                    fire_scatters(1)

                    @pl.when(i + 1 < NB // 2)
                    def _():
                        drain_scatters(0)
                        load_idx(b + 2, 0)
                        start_gather(0)
                    return carry
                lax.fori_loop(0, NB // 2, block2, 0)
                drain_scatters(0)
                drain_scatters(1)
                plsc.subcore_barrier()
                pltpu.sync_copy(acc.at[pl.ds(s * RPT, RPT)],
                                out_h.at[pl.ds(s * RPT, RPT), pl.ds(f * 32, 32)])

        tables = (xt_t, xt_t, xc_t)
        p = 0
        for o in range(3):
            for f in range(4):
                do_pass(p, f, tables[o], gx[o * 4 + f], sx[o], outs[o])
                p += 1

    out_type = [jax.ShapeDtypeStruct((NTP, 128), F32) for _ in range(3)]
    scratch = [
        pltpu.VMEM_SHARED((NTP, FCH), F32),
        pltpu.VMEM((EBLK,), jnp.int32),
        pltpu.VMEM((EBLK,), jnp.int32),
        pltpu.VMEM((NSB, 128), jnp.int32),
        pltpu.VMEM((NSB, 128), jnp.int32),
        pltpu.VMEM((EBLK, FCH), F32),
        pltpu.VMEM((EBLK, FCH), F32),
        pltpu.SemaphoreType.DMA,
        pltpu.SemaphoreType.DMA,
        pltpu.SemaphoreType.DMA,
        pltpu.SemaphoreType.DMA,
        pltpu.SemaphoreType.DMA,
    ]
    fn = pl.kernel(body, out_type=out_type, mesh=mesh, scratch_types=scratch,
                   compiler_params=pltpu.CompilerParams(use_tc_tiling_on_sc=False))
    return fn(xt32, xc32, *gidxs, *sidxs, zeros_h)


def kernel(x_target, x_context, edge_index_tt, edge_index_ct,
           W_lin_t, b_lin_t, W_lin_c, b_lin_c,
           W_self, b_self, W_s2d, b_s2d, W_d2s, b_d2s,
           W_ct_l, b_ct_l, W_ct_r, W_out, b_out):
    NT, D = x_target.shape
    H = W_lin_t.shape[1]
    E = edge_index_tt.shape[1]
    BM = 1000
    grid = (NT // BM,)
    RPT = (-(-(NT // NSUB) // ZROWS)) * ZROWS
    NTP = RPT * NSUB

    EPAD = ((E + NSUB * EBLK - 1) // (NSUB * EBLK)) * (NSUB * EBLK)
    npad = EPAD - E
    spread = jnp.arange(npad, dtype=jnp.int32) % 1024
    pad_g = spread * 4
    pad_s = NT + spread

    gbase = [jnp.concatenate([edge_index_tt[0] * 4, pad_g]),
             jnp.concatenate([edge_index_tt[1] * 4, pad_g]),
             jnp.concatenate([edge_index_ct[0] * 4, pad_g])]
    gidxs = [gbase[o] + f for o in range(3) for f in range(4)]

    def spad(a):
        return jnp.concatenate([a, pad_s]).reshape(EPAD // 128, 128)

    sidxs = [spad(edge_index_tt[1]), spad(edge_index_tt[0]), spad(edge_index_ct[1])]
    zeros_h = jnp.zeros((ZROWS, FCH), F32)
    ones_h = jnp.ones((EBLK, FCH), F32)

    cnt_a, cnt_b = _sc_count_call(NT, NTP, RPT, EPAD, sidxs, zeros_h, ones_h)

    b2 = lambda b: b.reshape(1, H)
    row_spec = pl.BlockSpec((BM, D), lambda i: (i, 0))
    w_spec = pl.BlockSpec((D, H), lambda i: (0, 0))
    b_spec = pl.BlockSpec((1, H), lambda i: (0, 0))

    tc1 = pl.pallas_call(
        _tc1_body,
        grid=grid,
        in_specs=[row_spec, row_spec, w_spec, b_spec, w_spec, b_spec,
                  w_spec, b_spec, b_spec, b_spec, b_spec, w_spec],
        out_specs=[row_spec] * 3,
        out_shape=[jax.ShapeDtypeStruct((NT, H), F32)] * 3,
    )
    xt, xc, pre = tc1(x_target, x_context, W_lin_t, b2(b_lin_t),
                      W_lin_c, b2(b_lin_c), W_self, b2(b_self),
                      b2(b_s2d), b2(b_d2s), b2(b_ct_l), W_ct_r)

    xt32 = xt.reshape(4 * NT, FCH)
    xc32 = xc.reshape(4 * NT, FCH)
    m1, m2, m3 = _sc_scatter_call(NT, NTP, RPT, EPAD, xt32, xc32, gidxs, sidxs,
                                  zeros_h)

    tc2 = pl.pallas_call(
        _tc2_body,
        grid=grid,
        in_specs=[row_spec] * 7 + [w_spec, w_spec, w_spec, w_spec, b_spec],
        out_specs=row_spec,
        out_shape=jax.ShapeDtypeStruct((NT, H), F32),
    )
    return tc2(pre, xt, m1, m2, m3, cnt_a, cnt_b,
               W_s2d, W_d2s, W_ct_l, W_out, b2(b_out))


# speedup vs baseline: 8.4940x; 1.0677x over previous
import jax
import jax.numpy as jnp
from jax import lax
from jax.experimental import pallas as pl
from jax.experimental.pallas import tpu as pltpu
from jax.experimental.pallas import tpu_sc as plsc

F32 = jnp.float32
FCH = 32
EBLK = 384
NSUB = 16
ZROWS = 800
NSB = EBLK // 128


def _tc1_body(xt_in, xc_in, wlt, blt, wlc, blc, wself, bself, bs2d, bd2s,
              bctl, wctr, xt_out, xc_out, pre_out):
    xt = jnp.maximum(jnp.dot(xt_in[:], wlt[:], preferred_element_type=F32) + blt[:], 0.0)
    xc = jnp.maximum(jnp.dot(xc_in[:], wlc[:], preferred_element_type=F32) + blc[:], 0.0)
    xt_out[:] = xt
    xc_out[:] = xc
    bias = bself[:] + 0.5 * bs2d[:] + 0.5 * bd2s[:] + bctl[:]
    pre_out[:] = jnp.dot(xt, wself[:] + wctr[:], preferred_element_type=F32) + bias


def _tc2_body(pre, xt, m1, m2, m3, ca, cb, ws2d, wd2s, wctl, wout, bout, out):
    d1 = jnp.maximum(ca[:][:, 0:1] + cb[:][:, 0:1], 1.0)
    d2 = jnp.maximum(ca[:][:, 8:9] + cb[:][:, 8:9], 1.0)
    d3 = jnp.maximum(ca[:][:, 16:17] + cb[:][:, 16:17], 1.0)
    h = (0.5 * pre[:]
         + 0.25 * jnp.dot(m1[:] / d1, ws2d[:], preferred_element_type=F32)
         + 0.25 * jnp.dot(m2[:] / d2, wd2s[:], preferred_element_type=F32)
         + 0.5 * jnp.dot(m3[:] / d3, wctl[:], preferred_element_type=F32))
    h = jnp.maximum(h + xt[:], 0.0)
    out[:] = jnp.dot(h, wout[:], preferred_element_type=F32) + bout[:]


def _zero_acc(acc, zeros_hbm, zsem, s, RPT):
    for k in range(RPT // ZROWS):
        pltpu.async_copy(zeros_hbm, acc.at[pl.ds(s * RPT + k * ZROWS, ZROWS)], zsem)
    for k in range(RPT // ZROWS):
        pltpu.make_async_copy(zeros_hbm, acc.at[pl.ds(s * RPT + k * ZROWS, ZROWS)],
                              zsem).wait()


def _sc_count_call(NT, NTP, RPT, EPAD, sidxs, zeros_h, ones_h):
    EH = EPAD // 2
    Q = EH // NSUB
    NB = Q // EBLK
    QR = Q // 128
    CW = 8
    mesh = plsc.VectorSubcoreMesh(core_axis_name="c", subcore_axis_name="s")

    def body(s1, s2, s3, zeros_hbm, ones_hbm, oa, ob, acc, idxs, rows,
             ssem, zsem):
        c = lax.axis_index("c")
        s = lax.axis_index("s")
        pltpu.sync_copy(ones_hbm, rows)

        for p, s_h in enumerate((s1, s2, s3)):
            for k in range(RPT // ZROWS):
                pltpu.async_copy(zeros_hbm.at[:, pl.ds(0, CW)],
                                 acc.at[pl.ds(s * RPT + k * ZROWS, ZROWS)], zsem)
            for k in range(RPT // ZROWS):
                pltpu.make_async_copy(zeros_hbm.at[:, pl.ds(0, CW)],
                                      acc.at[pl.ds(s * RPT + k * ZROWS, ZROWS)],
                                      zsem).wait()
            plsc.subcore_barrier()

            def cblock(b, carry):
                rowbase = c * (EH // 128) + s * QR + b * NSB
                pltpu.sync_copy(s_h.at[pl.ds(rowbase, NSB)], idxs)
                for j in range(NSB):
                    pltpu.async_copy(rows.at[pl.ds(j * 128, 128)],
                                     acc.at[idxs.at[j]], ssem, add=True)
                for j in range(NSB):
                    pltpu.make_async_copy(rows.at[pl.ds(j * 128, 128)],
                                          acc.at[idxs.at[j]], ssem).wait()
                return carry
            lax.fori_loop(0, NB, cblock, 0)
            plsc.subcore_barrier()

            @pl.when(c == 0)
            def _():
                pltpu.sync_copy(acc.at[pl.ds(s * RPT, RPT)],
                                oa.at[pl.ds(s * RPT, RPT), pl.ds(p * CW, CW)])

            @pl.when(c == 1)
            def _():
                pltpu.sync_copy(acc.at[pl.ds(s * RPT, RPT)],
                                ob.at[pl.ds(s * RPT, RPT), pl.ds(p * CW, CW)])

    out_type = [jax.ShapeDtypeStruct((NTP, 32), F32) for _ in range(2)]
    scratch = [
        pltpu.VMEM_SHARED((NTP, CW), F32),
        pltpu.VMEM((NSB, 128), jnp.int32),
        pltpu.VMEM((EBLK, CW), F32),
        pltpu.SemaphoreType.DMA,
        pltpu.SemaphoreType.DMA,
    ]
    fn = pl.kernel(body, out_type=out_type, mesh=mesh, scratch_types=scratch,
                   compiler_params=pltpu.CompilerParams(use_tc_tiling_on_sc=False))
    return fn(*sidxs, zeros_h, ones_h)


def _sc_scatter_call(NT, NTP, RPT, EPAD, xt32, xc32, gidxs, sidxs, zeros_h):
    Q = EPAD // NSUB
    NB = Q // EBLK
    QR = Q // 128
    mesh = plsc.VectorSubcoreMesh(core_axis_name="c", subcore_axis_name="s")

    def body(*refs):
        xt_t = refs[0]
        xc_t = refs[1]
        gx = refs[2:14]
        sx = refs[14:17]
        zeros_hbm = refs[17]
        outs = refs[18:21]
        acc = refs[21]
        idxg = refs[22:24]
        idxs = refs[24:26]
        rows = refs[26:28]
        sems = refs[28:30]
        ssems = refs[30:32]
        zsem = refs[32]

        c = lax.axis_index("c")
        s = lax.axis_index("s")

        def do_pass(p, f, table, g_h, s_h, out_h):
            @pl.when(c == (p % 2))
            def _():
                _zero_acc(acc, zeros_hbm, zsem, s, RPT)

                def load_idx(b, buf):
                    pltpu.sync_copy(s_h.at[pl.ds(s * QR + b * NSB, NSB)],
                                    idxs[buf])
                    pltpu.sync_copy(g_h.at[pl.ds(s * Q + b * EBLK, EBLK)],
                                    idxg[buf])

                def start_gather(buf):
                    pltpu.async_copy(table.at[idxg[buf]], rows[buf], sems[buf])

                def finish_gather(buf):
                    pltpu.make_async_copy(table.at[idxg[buf]], rows[buf],
                                          sems[buf]).wait()

                def fire_scatters(buf):
                    for j in range(NSB):
                        pltpu.async_copy(rows[buf].at[pl.ds(j * 128, 128)],
                                         acc.at[idxs[buf].at[j]], ssems[buf],
                                         add=True)

                def drain_scatters(buf):
                    for j in range(NSB):
                        pltpu.make_async_copy(rows[buf].at[pl.ds(j * 128, 128)],
                                              acc.at[idxs[buf].at[j]],
                                              ssems[buf]).wait()

                plsc.subcore_barrier()
                load_idx(0, 0)
                start_gather(0)

                def block2(i, carry):
                    b = i * 2

                    @pl.when(i > 0)
                    def _():
                        drain_scatters(1)
                    load_idx(b + 1, 1)
                    start_gather(1)
                    finish_gather(0)
                    fire_scatters(0)
                    finish_gather(1)
                    fire_scatters(1)

                    @pl.when(i + 1 < NB // 2)
                    def _():
                        drain_scatters(0)
                        load_idx(b + 2, 0)
                        start_gather(0)
                    return carry
                lax.fori_loop(0, NB // 2, block2, 0)
                drain_scatters(0)
                drain_scatters(1)
                plsc.subcore_barrier()
                pltpu.sync_copy(acc.at[pl.ds(s * RPT, RPT)],
                                out_h.at[pl.ds(s * RPT, RPT), pl.ds(f * 32, 32)])

        tables = (xt_t, xt_t, xc_t)
        p = 0
        for o in range(3):
            for f in range(4):
                do_pass(p, f, tables[o], gx[o * 4 + f], sx[o], outs[o])
                p += 1

    out_type = [jax.ShapeDtypeStruct((NTP, 128), F32) for _ in range(3)]
    scratch = [
        pltpu.VMEM_SHARED((NTP, FCH), F32),
        pltpu.VMEM((EBLK,), jnp.int32),
        pltpu.VMEM((EBLK,), jnp.int32),
        pltpu.VMEM((NSB, 128), jnp.int32),
        pltpu.VMEM((NSB, 128), jnp.int32),
        pltpu.VMEM((EBLK, FCH), F32),
        pltpu.VMEM((EBLK, FCH), F32),
        pltpu.SemaphoreType.DMA,
        pltpu.SemaphoreType.DMA,
        pltpu.SemaphoreType.DMA,
        pltpu.SemaphoreType.DMA,
        pltpu.SemaphoreType.DMA,
    ]
    fn = pl.kernel(body, out_type=out_type, mesh=mesh, scratch_types=scratch,
                   compiler_params=pltpu.CompilerParams(use_tc_tiling_on_sc=False))
    return fn(xt32, xc32, *gidxs, *sidxs, zeros_h)


def kernel(x_target, x_context, edge_index_tt, edge_index_ct,
           W_lin_t, b_lin_t, W_lin_c, b_lin_c,
           W_self, b_self, W_s2d, b_s2d, W_d2s, b_d2s,
           W_ct_l, b_ct_l, W_ct_r, W_out, b_out):
    NT, D = x_target.shape
    H = W_lin_t.shape[1]
    E = edge_index_tt.shape[1]
    BM = 1000
    grid = (NT // BM,)
    RPT = (-(-(NT // NSUB) // ZROWS)) * ZROWS
    NTP = RPT * NSUB

    EPAD = ((E + NSUB * EBLK - 1) // (NSUB * EBLK)) * (NSUB * EBLK)
    npad = EPAD - E
    spread = jnp.arange(npad, dtype=jnp.int32) % 1024
    pad_g = spread * 4
    pad_s = NT + spread

    gbase = [jnp.concatenate([edge_index_tt[0] * 4, pad_g]),
             jnp.concatenate([edge_index_tt[1] * 4, pad_g]),
             jnp.concatenate([edge_index_ct[0] * 4, pad_g])]
    gidxs = [gbase[o] + f for o in range(3) for f in range(4)]

    def spad(a):
        return jnp.concatenate([a, pad_s]).reshape(EPAD // 128, 128)

    sidxs = [spad(edge_index_tt[1]), spad(edge_index_tt[0]), spad(edge_index_ct[1])]
    zeros_h = jnp.zeros((ZROWS, FCH), F32)
    ones_h = jnp.ones((EBLK, 8), F32)

    cnt_a, cnt_b = _sc_count_call(NT, NTP, RPT, EPAD, sidxs, zeros_h, ones_h)

    b2 = lambda b: b.reshape(1, H)
    row_spec = pl.BlockSpec((BM, D), lambda i: (i, 0))
    w_spec = pl.BlockSpec((D, H), lambda i: (0, 0))
    b_spec = pl.BlockSpec((1, H), lambda i: (0, 0))

    tc1 = pl.pallas_call(
        _tc1_body,
        grid=grid,
        in_specs=[row_spec, row_spec, w_spec, b_spec, w_spec, b_spec,
                  w_spec, b_spec, b_spec, b_spec, b_spec, w_spec],
        out_specs=[row_spec] * 3,
        out_shape=[jax.ShapeDtypeStruct((NT, H), F32)] * 3,
    )
    xt, xc, pre = tc1(x_target, x_context, W_lin_t, b2(b_lin_t),
                      W_lin_c, b2(b_lin_c), W_self, b2(b_self),
                      b2(b_s2d), b2(b_d2s), b2(b_ct_l), W_ct_r)

    xt32 = xt.reshape(4 * NT, FCH)
    xc32 = xc.reshape(4 * NT, FCH)
    m1, m2, m3 = _sc_scatter_call(NT, NTP, RPT, EPAD, xt32, xc32, gidxs, sidxs,
                                  zeros_h)

    cnt_spec = pl.BlockSpec((BM, 32), lambda i: (i, 0))
    tc2 = pl.pallas_call(
        _tc2_body,
        grid=grid,
        in_specs=[row_spec] * 5 + [cnt_spec] * 2
        + [w_spec, w_spec, w_spec, w_spec, b_spec],
        out_specs=row_spec,
        out_shape=jax.ShapeDtypeStruct((NT, H), F32),
    )
    return tc2(pre, xt, m1, m2, m3, cnt_a, cnt_b,
               W_s2d, W_d2s, W_ct_l, W_out, b2(b_out))


# speedup vs baseline: 10.8110x; 1.2728x over previous
import jax
import jax.numpy as jnp
from jax import lax
from jax.experimental import pallas as pl
from jax.experimental.pallas import tpu as pltpu
from jax.experimental.pallas import tpu_sc as plsc

F32 = jnp.float32
FCH = 32
EBLK = 384
NSUB = 16
ZROWS = 800
NSB = EBLK // 128


def _tc1_body(xt_in, xc_in, wlt, blt, wlc, blc, wself, bself, bs2d, bd2s,
              bctl, wctr, xt_out, xtb_out, xcb_out, pre_out):
    xt = jnp.maximum(jnp.dot(xt_in[:], wlt[:], preferred_element_type=F32) + blt[:], 0.0)
    xc = jnp.maximum(jnp.dot(xc_in[:], wlc[:], preferred_element_type=F32) + blc[:], 0.0)
    xt_out[:] = xt
    xtb_out[:] = xt.astype(jnp.bfloat16)
    xcb_out[:] = xc.astype(jnp.bfloat16)
    bias = bself[:] + 0.5 * bs2d[:] + 0.5 * bd2s[:] + bctl[:]
    pre_out[:] = jnp.dot(xt, wself[:] + wctr[:], preferred_element_type=F32) + bias


def _tc2_body(pre, xt, m1, m2, m3, ca, cb, ws2d, wd2s, wctl, wout, bout, out):
    d1 = jnp.maximum(ca[:][:, 0:1] + cb[:][:, 0:1], 1.0)
    d2 = jnp.maximum(ca[:][:, 8:9] + cb[:][:, 8:9], 1.0)
    d3 = jnp.maximum(ca[:][:, 16:17] + cb[:][:, 16:17], 1.0)
    h = (0.5 * pre[:]
         + 0.25 * jnp.dot(m1[:].astype(F32) / d1, ws2d[:], preferred_element_type=F32)
         + 0.25 * jnp.dot(m2[:].astype(F32) / d2, wd2s[:], preferred_element_type=F32)
         + 0.5 * jnp.dot(m3[:].astype(F32) / d3, wctl[:], preferred_element_type=F32))
    h = jnp.maximum(h + xt[:], 0.0)
    out[:] = jnp.dot(h, wout[:], preferred_element_type=F32) + bout[:]


def _zero_acc(acc, zeros_hbm, zsem, s, RPT):
    for k in range(RPT // ZROWS):
        pltpu.async_copy(zeros_hbm, acc.at[pl.ds(s * RPT + k * ZROWS, ZROWS)], zsem)
    for k in range(RPT // ZROWS):
        pltpu.make_async_copy(zeros_hbm, acc.at[pl.ds(s * RPT + k * ZROWS, ZROWS)],
                              zsem).wait()


def _sc_count_call(NT, NTP, RPT, EPAD, sidxs, zeros_h, ones_h):
    EH = EPAD // 2
    Q = EH // NSUB
    NB = Q // EBLK
    QR = Q // 128
    CW = 8
    mesh = plsc.VectorSubcoreMesh(core_axis_name="c", subcore_axis_name="s")

    def body(s1, s2, s3, zeros_hbm, ones_hbm, oa, ob, acc, idxs, rows,
             ssem, zsem):
        c = lax.axis_index("c")
        s = lax.axis_index("s")
        pltpu.sync_copy(ones_hbm, rows)

        for p, s_h in enumerate((s1, s2, s3)):
            _zero_acc(acc, zeros_hbm, zsem, s, RPT)
            plsc.subcore_barrier()

            def cblock(b, carry):
                rowbase = c * (EH // 128) + s * QR + b * NSB
                pltpu.sync_copy(s_h.at[pl.ds(rowbase, NSB)], idxs)
                for j in range(NSB):
                    pltpu.async_copy(rows.at[pl.ds(j * 128, 128)],
                                     acc.at[idxs.at[j]], ssem, add=True)
                for j in range(NSB):
                    pltpu.make_async_copy(rows.at[pl.ds(j * 128, 128)],
                                          acc.at[idxs.at[j]], ssem).wait()
                return carry
            lax.fori_loop(0, NB, cblock, 0)
            plsc.subcore_barrier()

            @pl.when(c == 0)
            def _():
                pltpu.sync_copy(acc.at[pl.ds(s * RPT, RPT)],
                                oa.at[pl.ds(s * RPT, RPT), pl.ds(p * CW, CW)])

            @pl.when(c == 1)
            def _():
                pltpu.sync_copy(acc.at[pl.ds(s * RPT, RPT)],
                                ob.at[pl.ds(s * RPT, RPT), pl.ds(p * CW, CW)])

    out_type = [jax.ShapeDtypeStruct((NTP, 32), F32) for _ in range(2)]
    scratch = [
        pltpu.VMEM_SHARED((NTP, CW), F32),
        pltpu.VMEM((NSB, 128), jnp.int32),
        pltpu.VMEM((EBLK, CW), F32),
        pltpu.SemaphoreType.DMA,
        pltpu.SemaphoreType.DMA,
    ]
    fn = pl.kernel(body, out_type=out_type, mesh=mesh, scratch_types=scratch,
                   compiler_params=pltpu.CompilerParams(use_tc_tiling_on_sc=False))
    return fn(*sidxs, zeros_h, ones_h)


def _sc_scatter_call(NT, NTP, RPT, EPAD, xtb2, xcb2, gidxs, sidxs, zeros_h):
    Q = EPAD // NSUB
    NB = Q // EBLK
    QR = Q // 128
    mesh = plsc.VectorSubcoreMesh(core_axis_name="c", subcore_axis_name="s")

    def body(*refs):
        xt_t = refs[0]
        xc_t = refs[1]
        gx = refs[2:8]
        sx = refs[8:11]
        zeros_hbm = refs[11]
        outs = refs[12:15]
        acc = refs[15]
        idxg = refs[16:18]
        idxs = refs[18:20]
        rows = refs[20:22]
        sems = refs[22:24]
        ssems = refs[24:26]
        zsem = refs[26]

        c = lax.axis_index("c")
        s = lax.axis_index("s")

        def do_pass(p, f, table, g_h, s_h, out_h):
            @pl.when(c == (p % 2))
            def _():
                _zero_acc(acc, zeros_hbm, zsem, s, RPT)

                def load_idx(b, buf):
                    pltpu.sync_copy(s_h.at[pl.ds(s * QR + b * NSB, NSB)],
                                    idxs[buf])
                    pltpu.sync_copy(g_h.at[pl.ds(s * Q + b * EBLK, EBLK)],
                                    idxg[buf])

                def start_gather(buf):
                    pltpu.async_copy(table.at[idxg[buf]], rows[buf], sems[buf])

                def finish_gather(buf):
                    pltpu.make_async_copy(table.at[idxg[buf]], rows[buf],
                                          sems[buf]).wait()

                def fire_scatters(buf):
                    for j in range(NSB):
                        pltpu.async_copy(rows[buf].at[pl.ds(j * 128, 128)],
                                         acc.at[idxs[buf].at[j]], ssems[buf],
                                         add=True)

                def drain_scatters(buf):
                    for j in range(NSB):
                        pltpu.make_async_copy(rows[buf].at[pl.ds(j * 128, 128)],
                                              acc.at[idxs[buf].at[j]],
                                              ssems[buf]).wait()

                plsc.subcore_barrier()
                load_idx(0, 0)
                start_gather(0)

                def block2(i, carry):
                    b = i * 2

                    @pl.when(i > 0)
                    def _():
                        drain_scatters(1)
                    load_idx(b + 1, 1)
                    start_gather(1)
                    finish_gather(0)
                    fire_scatters(0)
                    finish_gather(1)
                    fire_scatters(1)

                    @pl.when(i + 1 < NB // 2)
                    def _():
                        drain_scatters(0)
                        load_idx(b + 2, 0)
                        start_gather(0)
                    return carry
                lax.fori_loop(0, NB // 2, block2, 0)
                drain_scatters(0)
                drain_scatters(1)
                plsc.subcore_barrier()
                pltpu.sync_copy(acc.at[pl.ds(s * RPT, RPT)],
                                out_h.at[pl.ds(s * RPT, RPT), pl.ds(f * 64, 64)])

        tables = (xt_t, xt_t, xc_t)
        p = 0
        for o in range(3):
            for f in range(2):
                do_pass(p, f, tables[o], gx[o * 2 + f], sx[o], outs[o])
                p += 1

    out_type = [jax.ShapeDtypeStruct((NTP, 128), jnp.bfloat16) for _ in range(3)]
    scratch = [
        pltpu.VMEM_SHARED((NTP, 64), jnp.bfloat16),
        pltpu.VMEM((EBLK,), jnp.int32),
        pltpu.VMEM((EBLK,), jnp.int32),
        pltpu.VMEM((NSB, 128), jnp.int32),
        pltpu.VMEM((NSB, 128), jnp.int32),
        pltpu.VMEM((EBLK, 64), jnp.bfloat16),
        pltpu.VMEM((EBLK, 64), jnp.bfloat16),
        pltpu.SemaphoreType.DMA,
        pltpu.SemaphoreType.DMA,
        pltpu.SemaphoreType.DMA,
        pltpu.SemaphoreType.DMA,
        pltpu.SemaphoreType.DMA,
    ]
    fn = pl.kernel(body, out_type=out_type, mesh=mesh, scratch_types=scratch,
                   compiler_params=pltpu.CompilerParams(use_tc_tiling_on_sc=False))
    return fn(xtb2, xcb2, *gidxs, *sidxs, zeros_h)


def kernel(x_target, x_context, edge_index_tt, edge_index_ct,
           W_lin_t, b_lin_t, W_lin_c, b_lin_c,
           W_self, b_self, W_s2d, b_s2d, W_d2s, b_d2s,
           W_ct_l, b_ct_l, W_ct_r, W_out, b_out):
    NT, D = x_target.shape
    H = W_lin_t.shape[1]
    E = edge_index_tt.shape[1]
    BM = 2000
    grid = (NT // BM,)
    RPT = (-(-(NT // NSUB) // ZROWS)) * ZROWS
    NTP = RPT * NSUB

    EPAD = ((E + NSUB * EBLK - 1) // (NSUB * EBLK)) * (NSUB * EBLK)
    npad = EPAD - E
    spread = jnp.arange(npad, dtype=jnp.int32) % 1024
    pad_g = spread * 2
    pad_s = NT + spread

    gbase = [jnp.concatenate([edge_index_tt[0] * 2, pad_g]),
             jnp.concatenate([edge_index_tt[1] * 2, pad_g]),
             jnp.concatenate([edge_index_ct[0] * 2, pad_g])]
    gidxs = [gbase[o] + f for o in range(3) for f in range(2)]

    def spad(a):
        return jnp.concatenate([a, pad_s]).reshape(EPAD // 128, 128)

    sidxs = [spad(edge_index_tt[1]), spad(edge_index_tt[0]), spad(edge_index_ct[1])]
    zeros_hc = jnp.zeros((ZROWS, 8), F32)
    zeros_hb = jnp.zeros((ZROWS, 64), jnp.bfloat16)
    ones_h = jnp.ones((EBLK, 8), F32)

    cnt_a, cnt_b = _sc_count_call(NT, NTP, RPT, EPAD, sidxs, zeros_hc, ones_h)

    b2 = lambda b: b.reshape(1, H)
    row_spec = pl.BlockSpec((BM, D), lambda i: (i, 0))
    w_spec = pl.BlockSpec((D, H), lambda i: (0, 0))
    b_spec = pl.BlockSpec((1, H), lambda i: (0, 0))

    tc1 = pl.pallas_call(
        _tc1_body,
        grid=grid,
        in_specs=[row_spec, row_spec, w_spec, b_spec, w_spec, b_spec,
                  w_spec, b_spec, b_spec, b_spec, b_spec, w_spec],
        out_specs=[row_spec] * 4,
        out_shape=[jax.ShapeDtypeStruct((NT, H), F32),
                   jax.ShapeDtypeStruct((NT, H), jnp.bfloat16),
                   jax.ShapeDtypeStruct((NT, H), jnp.bfloat16),
                   jax.ShapeDtypeStruct((NT, H), F32)],
    )
    xt, xtb, xcb, pre = tc1(x_target, x_context, W_lin_t, b2(b_lin_t),
                            W_lin_c, b2(b_lin_c), W_self, b2(b_self),
                            b2(b_s2d), b2(b_d2s), b2(b_ct_l), W_ct_r)

    xtb2 = xtb.reshape(2 * NT, 64)
    xcb2 = xcb.reshape(2 * NT, 64)
    m1, m2, m3 = _sc_scatter_call(NT, NTP, RPT, EPAD, xtb2, xcb2, gidxs, sidxs,
                                  zeros_hb)

    cnt_spec = pl.BlockSpec((BM, 32), lambda i: (i, 0))
    tc2 = pl.pallas_call(
        _tc2_body,
        grid=grid,
        in_specs=[row_spec] * 5 + [cnt_spec] * 2
        + [w_spec, w_spec, w_spec, w_spec, b_spec],
        out_specs=row_spec,
        out_shape=jax.ShapeDtypeStruct((NT, H), F32),
    )
    return tc2(pre, xt, m1, m2, m3, cnt_a, cnt_b,
               W_s2d, W_d2s, W_ct_l, W_out, b2(b_out))
